# baseline (speedup 1.0000x reference)
import functools

import jax
import jax.numpy as jnp
from jax import lax
from jax.experimental import pallas as pl
from jax.experimental.pallas import tpu as pltpu
from jax.experimental.pallas import tpu_sc as plsc

NC = 2
NS = 16
NW = NC * NS
CH = 128
OFF = 499968
NP = OFF + 64
P2 = 249984
NR = NP - P2
CBLK = 8064


def _round_bf16_bits(v):
    u = jax.lax.bitcast_convert_type(v, jnp.uint32)
    return (u + jnp.uint32(0x8000)) & jnp.uint32(0xFFFF0000)


def _repack_body(ta_ref, tb_ref, tc_ref, td_ref, eye_ref, out_ref):
    dn = (((0,), (0,)), ((), ()))
    mm = lambda t: lax.dot_general(t[...], eye_ref[...], dn,
                                   preferred_element_type=jnp.float32)
    lo = jnp.concatenate([mm(ta_ref), mm(tb_ref)], axis=1)
    hi = jnp.concatenate([mm(tc_ref), mm(td_ref)], axis=1)
    packed = (_round_bf16_bits(lo) >> 16) | _round_bf16_bits(hi)
    out_ref[...] = jax.lax.bitcast_convert_type(packed, jnp.float32)


def _repack(embT):
    H = embT.shape[0]
    nblk = (NR + CBLK - 1) // CBLK
    eye = jnp.eye(H, dtype=jnp.float32)
    bA, bB, bC, bD = 0, OFF // CBLK, P2 // CBLK, (P2 + OFF) // CBLK
    return pl.pallas_call(
        _repack_body,
        grid=(nblk,),
        in_specs=[
            pl.BlockSpec((H, CBLK), lambda i: (0, i + bA)),
            pl.BlockSpec((H, CBLK), lambda i: (0, i + bB)),
            pl.BlockSpec((H, CBLK), lambda i: (0, i + bC)),
            pl.BlockSpec((H, CBLK), lambda i: (0, i + bD)),
            pl.BlockSpec((H, H), lambda i: (0, 0)),
        ],
        out_specs=pl.BlockSpec((CBLK, 2 * H), lambda i: (i, 0)),
        out_shape=jax.ShapeDtypeStruct((NR, 2 * H), jnp.float32),
    )(embT, embT, embT, embT, eye)


def _sc_gather(idx3, table, n_ch, per_w):
    mesh = plsc.VectorSubcoreMesh(
        core_axis_name="c", subcore_axis_name="s",
        num_cores=NC, num_subcores=NS)

    @functools.partial(
        pl.kernel,
        out_type=jax.ShapeDtypeStruct((NW * per_w, 128), jnp.float32),
        mesh=mesh,
        scratch_types=[
            pltpu.VMEM((n_ch, CH), jnp.int32),
            pltpu.VMEM((2, CH, 128), jnp.float32),
            pltpu.SemaphoreType.DMA,
            pltpu.SemaphoreType.DMA,
        ],
    )
    def body(idx_hbm, table_hbm, out_hbm, idx_v, rows_v, sem_g, sem_w):
        wid = lax.axis_index("s") * NC + lax.axis_index("c")
        base = wid * per_w
        pltpu.sync_copy(idx_hbm.at[wid], idx_v)
        writes = [None, None]
        for j in range(n_ch):
            s = j % 2
            if writes[s] is not None:
                writes[s].wait()
            pltpu.async_copy(
                table_hbm.at[idx_v.at[j]], rows_v.at[s], sem_g
            ).wait()
            writes[s] = pltpu.async_copy(
                rows_v.at[s], out_hbm.at[pl.ds(base + j * CH, CH)], sem_w
            )
        for w in writes:
            if w is not None:
                w.wait()

    return body(idx3, table)


def _mlp_body(ga_ref, gb_ref, fa_ref, fb_ref, wa_ref, wb_ref, b1_ref,
              w2_ref, b2_ref, o_ref):
    bb = ga_ref.shape[1]
    ge64 = lax.broadcasted_iota(jnp.int32, (bb, 128), 1) >= 64

    def unpack_select(g_ref, f_ref):
        u = jax.lax.bitcast_convert_type(g_ref[0], jnp.uint32)
        lo = jax.lax.bitcast_convert_type(u << 16, jnp.float32)
        hi = jax.lax.bitcast_convert_type(u & jnp.uint32(0xFFFF0000),
                                          jnp.float32)
        f = f_ref[0]
        p_pack = f[:, 0:1]
        p_half = f[:, 1:2]
        v = jnp.where(p_pack > 0.5, hi, lo)
        m = jnp.where(ge64, p_half, 1.0 - p_half)
        return v * m

    am = unpack_select(ga_ref, fa_ref)
    bm = unpack_select(gb_ref, fb_ref)
    h = jnp.dot(am, wa_ref[...], preferred_element_type=jnp.float32)
    h = h + jnp.dot(bm, wb_ref[...], preferred_element_type=jnp.float32)
    h = jnp.maximum(h + b1_ref[...], 0.0)
    o_ref[...] = (
        jnp.dot(h, w2_ref[...], preferred_element_type=jnp.float32)
        + b2_ref[...]
    )


def kernel(x, emb, W1, b1, W2, b2):
    B = x.shape[0]
    H = emb.shape[1]
    L = W2.shape[0]

    total = 2 * B
    per_w = total // NW
    n_ch = per_w // CH

    xt = x.T
    half = (xt >= NP).astype(jnp.int32)
    q = xt - OFF * half
    packhi = (q >= NR).astype(jnp.int32)
    idx3 = (q - P2 * packhi).reshape(NW, n_ch, CH)
    flags = jnp.stack(
        [packhi.astype(jnp.float32), half.astype(jnp.float32)], axis=-1
    )

    embT = emb.T
    Q = _repack(embT)
    g = _sc_gather(idx3, Q, n_ch, per_w)
    g3 = g.reshape(2, B, 2 * H)

    Wa = W1[:, :H].T
    Wb = W1[:, H:].T
    WaS = jnp.concatenate([Wa, Wa], axis=0)
    WbS = jnp.concatenate([Wb, Wb], axis=0)
    W2T = W2.T

    BB = 2048
    grid = (B // BB,)
    out = pl.pallas_call(
        _mlp_body,
        grid=grid,
        in_specs=[
            pl.BlockSpec((1, BB, 2 * H), lambda i: (0, i, 0)),
            pl.BlockSpec((1, BB, 2 * H), lambda i: (1, i, 0)),
            pl.BlockSpec((1, BB, 2), lambda i: (0, i, 0)),
            pl.BlockSpec((1, BB, 2), lambda i: (1, i, 0)),
            pl.BlockSpec((2 * H, H), lambda i: (0, 0)),
            pl.BlockSpec((2 * H, H), lambda i: (0, 0)),
            pl.BlockSpec((1, H), lambda i: (0, 0)),
            pl.BlockSpec((H, L), lambda i: (0, 0)),
            pl.BlockSpec((1, L), lambda i: (0, 0)),
        ],
        out_specs=pl.BlockSpec((BB, L), lambda i: (i, 0)),
        out_shape=jax.ShapeDtypeStruct((B, L), jnp.float32),
    )(g3, g3, flags, flags, WaS, WbS, b1.reshape(1, H), W2T, b2.reshape(1, L))
    return out


# speedup vs baseline: 1.4021x; 1.4021x over previous
import functools

import jax
import jax.numpy as jnp
from jax import lax
from jax.experimental import pallas as pl
from jax.experimental.pallas import tpu as pltpu
from jax.experimental.pallas import tpu_sc as plsc

NC = 2
NS = 16
NW = NC * NS
CH = 128
OFF = 499968
NP = OFF + 64
P2 = 249984
NR = NP - P2
CBLK = 8064


def _round_bf16_bits(v):
    u = jax.lax.bitcast_convert_type(v, jnp.uint32)
    return (u + jnp.uint32(0x8000)) & jnp.uint32(0xFFFF0000)


def _repack_body(ta_ref, tb_ref, tc_ref, td_ref, eye_ref, out_ref):
    dn = (((0,), (0,)), ((), ()))
    lo = lax.dot_general(
        jnp.concatenate([ta_ref[...], tb_ref[...]], axis=0),
        eye_ref[...], dn, preferred_element_type=jnp.float32)
    hi = lax.dot_general(
        jnp.concatenate([tc_ref[...], td_ref[...]], axis=0),
        eye_ref[...], dn, preferred_element_type=jnp.float32)
    packed = (_round_bf16_bits(lo) >> 16) | _round_bf16_bits(hi)
    out_ref[...] = jax.lax.bitcast_convert_type(packed, jnp.float32)


def _repack(embT):
    H = embT.shape[0]
    nblk = (NR + CBLK - 1) // CBLK
    eye = jnp.eye(2 * H, dtype=jnp.float32)
    bA, bB, bC, bD = 0, OFF // CBLK, P2 // CBLK, (P2 + OFF) // CBLK
    return pl.pallas_call(
        _repack_body,
        grid=(nblk,),
        in_specs=[
            pl.BlockSpec((H, CBLK), lambda i: (0, i + bA)),
            pl.BlockSpec((H, CBLK), lambda i: (0, i + bB)),
            pl.BlockSpec((H, CBLK), lambda i: (0, i + bC)),
            pl.BlockSpec((H, CBLK), lambda i: (0, i + bD)),
            pl.BlockSpec((2 * H, 2 * H), lambda i: (0, 0)),
        ],
        out_specs=pl.BlockSpec((CBLK, 2 * H), lambda i: (i, 0)),
        out_shape=jax.ShapeDtypeStruct((NR, 2 * H), jnp.float32),
        compiler_params=pltpu.CompilerParams(
            fuse_transposed_lhs_in_matmul=True),
    )(embT, embT, embT, embT, eye)


def _sc_gather(idx3, table, n_ch, per_w):
    mesh = plsc.VectorSubcoreMesh(
        core_axis_name="c", subcore_axis_name="s",
        num_cores=NC, num_subcores=NS)

    @functools.partial(
        pl.kernel,
        out_type=jax.ShapeDtypeStruct((NW * per_w, 128), jnp.float32),
        mesh=mesh,
        scratch_types=[
            pltpu.VMEM((n_ch, CH), jnp.int32),
            pltpu.VMEM((2, CH, 128), jnp.float32),
            pltpu.SemaphoreType.DMA,
            pltpu.SemaphoreType.DMA,
        ],
    )
    def body(idx_hbm, table_hbm, out_hbm, idx_v, rows_v, sem_g, sem_w):
        wid = lax.axis_index("s") * NC + lax.axis_index("c")
        base = wid * per_w
        pltpu.sync_copy(idx_hbm.at[wid], idx_v)
        writes = [None, None]
        for j in range(n_ch):
            s = j % 2
            if writes[s] is not None:
                writes[s].wait()
            pltpu.async_copy(
                table_hbm.at[idx_v.at[j]], rows_v.at[s], sem_g
            ).wait()
            writes[s] = pltpu.async_copy(
                rows_v.at[s], out_hbm.at[pl.ds(base + j * CH, CH)], sem_w
            )
        for w in writes:
            if w is not None:
                w.wait()

    return body(idx3, table)


def _mlp_body(ga_ref, gb_ref, fa_ref, fb_ref, wa_ref, wb_ref, b1_ref,
              w2_ref, b2_ref, o_ref):
    bb = ga_ref.shape[1]
    ge64 = lax.broadcasted_iota(jnp.int32, (bb, 128), 1) >= 64

    def unpack_select(g_ref, f_ref):
        u = jax.lax.bitcast_convert_type(g_ref[0], jnp.uint32)
        lo = jax.lax.bitcast_convert_type(u << 16, jnp.float32)
        hi = jax.lax.bitcast_convert_type(u & jnp.uint32(0xFFFF0000),
                                          jnp.float32)
        f = f_ref[0]
        p_pack = f[:, 0:1]
        p_half = f[:, 1:2]
        v = jnp.where(p_pack > 0.5, hi, lo)
        m = jnp.where(ge64, p_half, 1.0 - p_half)
        return v * m

    am = unpack_select(ga_ref, fa_ref)
    bm = unpack_select(gb_ref, fb_ref)
    h = jnp.dot(am, wa_ref[...], preferred_element_type=jnp.float32)
    h = h + jnp.dot(bm, wb_ref[...], preferred_element_type=jnp.float32)
    h = jnp.maximum(h + b1_ref[...], 0.0)
    o_ref[...] = (
        jnp.dot(h, w2_ref[...], preferred_element_type=jnp.float32)
        + b2_ref[...]
    )


def kernel(x, emb, W1, b1, W2, b2):
    B = x.shape[0]
    H = emb.shape[1]
    L = W2.shape[0]

    total = 2 * B
    per_w = total // NW
    n_ch = per_w // CH

    xt = x.T
    half = (xt >= NP).astype(jnp.int32)
    q = xt - OFF * half
    packhi = (q >= NR).astype(jnp.int32)
    idx3 = (q - P2 * packhi).reshape(NW, n_ch, CH)
    flags = jnp.stack(
        [packhi.astype(jnp.float32), half.astype(jnp.float32)], axis=-1
    )

    embT = emb.T
    Q = _repack(embT)
    g = _sc_gather(idx3, Q, n_ch, per_w)
    g3 = g.reshape(2, B, 2 * H)

    Wa = W1[:, :H].T
    Wb = W1[:, H:].T
    WaS = jnp.concatenate([Wa, Wa], axis=0)
    WbS = jnp.concatenate([Wb, Wb], axis=0)
    W2T = W2.T

    BB = 2048
    grid = (B // BB,)
    out = pl.pallas_call(
        _mlp_body,
        grid=grid,
        in_specs=[
            pl.BlockSpec((1, BB, 2 * H), lambda i: (0, i, 0)),
            pl.BlockSpec((1, BB, 2 * H), lambda i: (1, i, 0)),
            pl.BlockSpec((1, BB, 2), lambda i: (0, i, 0)),
            pl.BlockSpec((1, BB, 2), lambda i: (1, i, 0)),
            pl.BlockSpec((2 * H, H), lambda i: (0, 0)),
            pl.BlockSpec((2 * H, H), lambda i: (0, 0)),
            pl.BlockSpec((1, H), lambda i: (0, 0)),
            pl.BlockSpec((H, L), lambda i: (0, 0)),
            pl.BlockSpec((1, L), lambda i: (0, 0)),
        ],
        out_specs=pl.BlockSpec((BB, L), lambda i: (i, 0)),
        out_shape=jax.ShapeDtypeStruct((B, L), jnp.float32),
    )(g3, g3, flags, flags, WaS, WbS, b1.reshape(1, H), W2T, b2.reshape(1, L))
    return out


# speedup vs baseline: 1.6292x; 1.1620x over previous
import functools

import jax
import jax.numpy as jnp
from jax import lax
from jax.experimental import pallas as pl
from jax.experimental.pallas import tpu as pltpu
from jax.experimental.pallas import tpu_sc as plsc

NC = 2
NS = 16
NW = NC * NS
CH = 128
OFF = 499968
NP = OFF + 64
P2 = 249984
NR = NP - P2
CBLK = 8064


def _round_bf16_bits(v):
    u = jax.lax.bitcast_convert_type(v, jnp.uint32)
    return (u + jnp.uint32(0x8000)) & jnp.uint32(0xFFFF0000)


def _repack_body(ta_ref, tb_ref, tc_ref, td_ref, eye_ref, out_ref):
    dn = (((0,), (0,)), ((), ()))
    lo = lax.dot_general(
        jnp.concatenate([ta_ref[...], tb_ref[...]], axis=0),
        eye_ref[...], dn, preferred_element_type=jnp.float32)
    hi = lax.dot_general(
        jnp.concatenate([tc_ref[...], td_ref[...]], axis=0),
        eye_ref[...], dn, preferred_element_type=jnp.float32)
    packed = (_round_bf16_bits(lo) >> 16) | _round_bf16_bits(hi)
    out_ref[...] = jax.lax.bitcast_convert_type(packed, jnp.float32)


def _repack(embT):
    H = embT.shape[0]
    nblk = (NR + CBLK - 1) // CBLK
    eye = jnp.eye(2 * H, dtype=jnp.float32)
    bA, bB, bC, bD = 0, OFF // CBLK, P2 // CBLK, (P2 + OFF) // CBLK
    return pl.pallas_call(
        _repack_body,
        grid=(nblk,),
        in_specs=[
            pl.BlockSpec((H, CBLK), lambda i: (0, i + bA)),
            pl.BlockSpec((H, CBLK), lambda i: (0, i + bB)),
            pl.BlockSpec((H, CBLK), lambda i: (0, i + bC)),
            pl.BlockSpec((H, CBLK), lambda i: (0, i + bD)),
            pl.BlockSpec((2 * H, 2 * H), lambda i: (0, 0)),
        ],
        out_specs=pl.BlockSpec((CBLK, 2 * H), lambda i: (i, 0)),
        out_shape=jax.ShapeDtypeStruct((NR, 2 * H), jnp.float32),
        compiler_params=pltpu.CompilerParams(
            fuse_transposed_lhs_in_matmul=True),
    )(embT, embT, embT, embT, eye)


def _sc_gather(idx3, table, n_ch, per_w):
    mesh = plsc.VectorSubcoreMesh(
        core_axis_name="c", subcore_axis_name="s",
        num_cores=NC, num_subcores=NS)

    @functools.partial(
        pl.kernel,
        out_type=jax.ShapeDtypeStruct((NW * per_w, 128), jnp.float32),
        mesh=mesh,
        scratch_types=[
            pltpu.VMEM((n_ch, CH), jnp.int32),
            pltpu.VMEM((2, CH, 128), jnp.float32),
            pltpu.SemaphoreType.DMA,
            pltpu.SemaphoreType.DMA,
        ],
    )
    def body(idx_hbm, table_hbm, out_hbm, idx_v, rows_v, sem_g, sem_w):
        wid = lax.axis_index("s") * NC + lax.axis_index("c")
        base = wid * per_w
        pltpu.sync_copy(idx_hbm.at[wid], idx_v)
        writes = [None, None]
        for j in range(n_ch):
            s = j % 2
            if writes[s] is not None:
                writes[s].wait()
            pltpu.async_copy(
                table_hbm.at[idx_v.at[j]], rows_v.at[s], sem_g
            ).wait()
            writes[s] = pltpu.async_copy(
                rows_v.at[s], out_hbm.at[pl.ds(base + j * CH, CH)], sem_w
            )
        for w in writes:
            if w is not None:
                w.wait()

    return body(idx3, table)


def _mlp_body(ga_ref, gb_ref, fa_ref, fb_ref, wa_ref, wb_ref, b1_ref,
              w2_ref, b2_ref, o_ref):
    bb = ga_ref.shape[1]
    ge64 = lax.broadcasted_iota(jnp.int32, (bb, 128), 1) >= 64
    ones = jnp.ones((1, 128), jnp.float32)
    dn0 = (((0,), (0,)), ((), ()))

    def unpack_select(g_ref, f_ref):
        u = jax.lax.bitcast_convert_type(g_ref[0], jnp.uint32)
        lo = jax.lax.bitcast_convert_type(u << 16, jnp.float32)
        hi = jax.lax.bitcast_convert_type(u & jnp.uint32(0xFFFF0000),
                                          jnp.float32)
        f = f_ref[0]
        mp = lax.dot_general(f[0:1, :], ones, dn0,
                             preferred_element_type=jnp.float32)
        mh = lax.dot_general(f[1:2, :], ones, dn0,
                             preferred_element_type=jnp.float32)
        v = jnp.where(mp > 0.5, hi, lo)
        m = jnp.where(ge64, mh, 1.0 - mh)
        return v * m

    am = unpack_select(ga_ref, fa_ref)
    bm = unpack_select(gb_ref, fb_ref)
    was = jnp.concatenate([wa_ref[...], wa_ref[...]], axis=0)
    wbs = jnp.concatenate([wb_ref[...], wb_ref[...]], axis=0)
    h = jnp.dot(am, was, preferred_element_type=jnp.float32)
    h = h + jnp.dot(bm, wbs, preferred_element_type=jnp.float32)
    h = jnp.maximum(h + b1_ref[...], 0.0)
    o_ref[...] = (
        jnp.dot(h, w2_ref[...], preferred_element_type=jnp.float32)
        + b2_ref[...]
    )


def kernel(x, emb, W1, b1, W2, b2):
    B = x.shape[0]
    H = emb.shape[1]
    L = W2.shape[0]

    total = 2 * B
    per_w = total // NW
    n_ch = per_w // CH

    xt = x.T
    half = (xt >= NP).astype(jnp.int32)
    q = xt - OFF * half
    packhi = (q >= NR).astype(jnp.int32)
    idx3 = (q - P2 * packhi).reshape(NW, n_ch, CH)
    flags = jnp.stack(
        [packhi.astype(jnp.float32), half.astype(jnp.float32)], axis=1
    )

    embT = emb.T
    Q = _repack(embT)
    g = _sc_gather(idx3, Q, n_ch, per_w)
    g3 = g.reshape(2, B, 2 * H)

    Wa = W1[:, :H].T
    Wb = W1[:, H:].T
    W2T = W2.T

    BB = 2048
    grid = (B // BB,)
    out = pl.pallas_call(
        _mlp_body,
        grid=grid,
        in_specs=[
            pl.BlockSpec((1, BB, 2 * H), lambda i: (0, i, 0)),
            pl.BlockSpec((1, BB, 2 * H), lambda i: (1, i, 0)),
            pl.BlockSpec((1, 2, BB), lambda i: (0, 0, i)),
            pl.BlockSpec((1, 2, BB), lambda i: (1, 0, i)),
            pl.BlockSpec((H, H), lambda i: (0, 0)),
            pl.BlockSpec((H, H), lambda i: (0, 0)),
            pl.BlockSpec((1, H), lambda i: (0, 0)),
            pl.BlockSpec((H, L), lambda i: (0, 0)),
            pl.BlockSpec((1, L), lambda i: (0, 0)),
        ],
        out_specs=pl.BlockSpec((BB, L), lambda i: (i, 0)),
        out_shape=jax.ShapeDtypeStruct((B, L), jnp.float32),
    )(g3, g3, flags, flags, Wa, Wb, b1.reshape(1, H), W2T, b2.reshape(1, L))
    return out


# speedup vs baseline: 1.6472x; 1.0110x over previous
import functools

import jax
import jax.numpy as jnp
from jax import lax
from jax.experimental import pallas as pl
from jax.experimental.pallas import tpu as pltpu
from jax.experimental.pallas import tpu_sc as plsc

NC = 2
NS = 16
NW = NC * NS
CH = 128
OFF = 499968
NP = OFF + 64
P2 = 249984
NR = NP - P2
CBLK = 11904


def _round_bf16_bits(v):
    u = jax.lax.bitcast_convert_type(v, jnp.uint32)
    return (u + jnp.uint32(0x8000)) & jnp.uint32(0xFFFF0000)


def _repack_body(ta_ref, tb_ref, tc_ref, td_ref, eye_ref, out_ref):
    dn = (((0,), (0,)), ((), ()))
    lo = lax.dot_general(
        jnp.concatenate([ta_ref[...], tb_ref[...]], axis=0),
        eye_ref[...], dn, preferred_element_type=jnp.float32)
    hi = lax.dot_general(
        jnp.concatenate([tc_ref[...], td_ref[...]], axis=0),
        eye_ref[...], dn, preferred_element_type=jnp.float32)
    packed = (_round_bf16_bits(lo) >> 16) | _round_bf16_bits(hi)
    out_ref[...] = jax.lax.bitcast_convert_type(packed, jnp.float32)


def _repack(embT):
    H = embT.shape[0]
    nblk = (NR + CBLK - 1) // CBLK
    eye = jnp.eye(2 * H, dtype=jnp.float32)
    bA, bB, bC, bD = 0, OFF // CBLK, P2 // CBLK, (P2 + OFF) // CBLK
    return pl.pallas_call(
        _repack_body,
        grid=(nblk,),
        in_specs=[
            pl.BlockSpec((H, CBLK), lambda i: (0, i + bA)),
            pl.BlockSpec((H, CBLK), lambda i: (0, i + bB)),
            pl.BlockSpec((H, CBLK), lambda i: (0, i + bC)),
            pl.BlockSpec((H, CBLK), lambda i: (0, i + bD)),
            pl.BlockSpec((2 * H, 2 * H), lambda i: (0, 0)),
        ],
        out_specs=pl.BlockSpec((CBLK, 2 * H), lambda i: (i, 0)),
        out_shape=jax.ShapeDtypeStruct((NR, 2 * H), jnp.float32),
        compiler_params=pltpu.CompilerParams(
            fuse_transposed_lhs_in_matmul=True),
    )(embT, embT, embT, embT, eye)


def _sc_gather(idx3, table, n_ch, per_w):
    mesh = plsc.VectorSubcoreMesh(
        core_axis_name="c", subcore_axis_name="s",
        num_cores=NC, num_subcores=NS)

    @functools.partial(
        pl.kernel,
        out_type=jax.ShapeDtypeStruct((NW * per_w, 128), jnp.float32),
        mesh=mesh,
        scratch_types=[
            pltpu.VMEM((n_ch, CH), jnp.int32),
            pltpu.VMEM((2, CH, 128), jnp.float32),
            pltpu.SemaphoreType.DMA,
            pltpu.SemaphoreType.DMA,
        ],
    )
    def body(idx_hbm, table_hbm, out_hbm, idx_v, rows_v, sem_g, sem_w):
        wid = lax.axis_index("s") * NC + lax.axis_index("c")
        base = wid * per_w
        pltpu.sync_copy(idx_hbm.at[wid], idx_v)
        writes = [None, None]
        for j in range(n_ch):
            s = j % 2
            if writes[s] is not None:
                writes[s].wait()
            pltpu.async_copy(
                table_hbm.at[idx_v.at[j]], rows_v.at[s], sem_g
            ).wait()
            writes[s] = pltpu.async_copy(
                rows_v.at[s], out_hbm.at[pl.ds(base + j * CH, CH)], sem_w
            )
        for w in writes:
            if w is not None:
                w.wait()

    return body(idx3, table)


def _mlp_body(ga_ref, gb_ref, fa_ref, fb_ref, wa_ref, wb_ref, b1_ref,
              w2_ref, b2_ref, o_ref):
    bb = ga_ref.shape[1]
    ge64 = lax.broadcasted_iota(jnp.int32, (bb, 128), 1) >= 64
    ones = jnp.ones((1, 128), jnp.float32)
    dn0 = (((0,), (0,)), ((), ()))

    def unpack_select(g_ref, f_ref):
        u = jax.lax.bitcast_convert_type(g_ref[0], jnp.uint32)
        lo = jax.lax.bitcast_convert_type(u << 16, jnp.float32)
        hi = jax.lax.bitcast_convert_type(u & jnp.uint32(0xFFFF0000),
                                          jnp.float32)
        f = f_ref[0]
        mp = lax.dot_general(f[0:1, :], ones, dn0,
                             preferred_element_type=jnp.float32)
        mh = lax.dot_general(f[1:2, :], ones, dn0,
                             preferred_element_type=jnp.float32)
        v = jnp.where(mp > 0.5, hi, lo)
        m = jnp.where(ge64, mh, 1.0 - mh)
        return v * m

    am = unpack_select(ga_ref, fa_ref)
    bm = unpack_select(gb_ref, fb_ref)
    was = jnp.concatenate([wa_ref[...], wa_ref[...]], axis=0)
    wbs = jnp.concatenate([wb_ref[...], wb_ref[...]], axis=0)
    h = jnp.dot(am, was, preferred_element_type=jnp.float32)
    h = h + jnp.dot(bm, wbs, preferred_element_type=jnp.float32)
    h = jnp.maximum(h + b1_ref[...], 0.0)
    o_ref[...] = (
        jnp.dot(h, w2_ref[...], preferred_element_type=jnp.float32)
        + b2_ref[...]
    )


def kernel(x, emb, W1, b1, W2, b2):
    B = x.shape[0]
    H = emb.shape[1]
    L = W2.shape[0]

    total = 2 * B
    per_w = total // NW
    n_ch = per_w // CH

    xt = x.T
    half = (xt >= NP).astype(jnp.int32)
    q = xt - OFF * half
    packhi = (q >= NR).astype(jnp.int32)
    idx3 = (q - P2 * packhi).reshape(NW, n_ch, CH)
    flags = jnp.stack(
        [packhi.astype(jnp.float32), half.astype(jnp.float32)], axis=1
    )

    embT = emb.T
    Q = _repack(embT)
    g = _sc_gather(idx3, Q, n_ch, per_w)
    g3 = g.reshape(2, B, 2 * H)

    Wa = W1[:, :H].T
    Wb = W1[:, H:].T
    W2T = W2.T

    BB = 2048
    grid = (B // BB,)
    out = pl.pallas_call(
        _mlp_body,
        grid=grid,
        in_specs=[
            pl.BlockSpec((1, BB, 2 * H), lambda i: (0, i, 0)),
            pl.BlockSpec((1, BB, 2 * H), lambda i: (1, i, 0)),
            pl.BlockSpec((1, 2, BB), lambda i: (0, 0, i)),
            pl.BlockSpec((1, 2, BB), lambda i: (1, 0, i)),
            pl.BlockSpec((H, H), lambda i: (0, 0)),
            pl.BlockSpec((H, H), lambda i: (0, 0)),
            pl.BlockSpec((1, H), lambda i: (0, 0)),
            pl.BlockSpec((H, L), lambda i: (0, 0)),
            pl.BlockSpec((1, L), lambda i: (0, 0)),
        ],
        out_specs=pl.BlockSpec((BB, L), lambda i: (i, 0)),
        out_shape=jax.ShapeDtypeStruct((B, L), jnp.float32),
    )(g3, g3, flags, flags, Wa, Wb, b1.reshape(1, H), W2T, b2.reshape(1, L))
    return out


# speedup vs baseline: 1.6554x; 1.0050x over previous
import functools

import jax
import jax.numpy as jnp
from jax import lax
from jax.experimental import pallas as pl
from jax.experimental.pallas import tpu as pltpu
from jax.experimental.pallas import tpu_sc as plsc

NC = 2
NS = 16
NW = NC * NS
CH = 128
OFF = 499968
NP = OFF + 64
P2 = 249984
NR = NP - P2
CBLK = 11904


def _round_bf16_bits(v):
    u = jax.lax.bitcast_convert_type(v, jnp.uint32)
    return (u + jnp.uint32(0x8000)) & jnp.uint32(0xFFFF0000)


def _repack_body(ta_ref, tb_ref, tc_ref, td_ref, eye_ref, out_ref):
    dn = (((0,), (0,)), ((), ()))
    lo = lax.dot_general(
        jnp.concatenate([ta_ref[...], tb_ref[...]], axis=0),
        eye_ref[...], dn, preferred_element_type=jnp.float32)
    hi = lax.dot_general(
        jnp.concatenate([tc_ref[...], td_ref[...]], axis=0),
        eye_ref[...], dn, preferred_element_type=jnp.float32)
    packed = (_round_bf16_bits(lo) >> 16) | _round_bf16_bits(hi)
    out_ref[...] = jax.lax.bitcast_convert_type(packed, jnp.float32)


def _repack(embT):
    H = embT.shape[0]
    nblk = (NR + CBLK - 1) // CBLK
    eye = jnp.eye(2 * H, dtype=jnp.float32)
    bA, bB, bC, bD = 0, OFF // CBLK, P2 // CBLK, (P2 + OFF) // CBLK
    return pl.pallas_call(
        _repack_body,
        grid=(nblk,),
        in_specs=[
            pl.BlockSpec((H, CBLK), lambda i: (0, i + bA)),
            pl.BlockSpec((H, CBLK), lambda i: (0, i + bB)),
            pl.BlockSpec((H, CBLK), lambda i: (0, i + bC)),
            pl.BlockSpec((H, CBLK), lambda i: (0, i + bD)),
            pl.BlockSpec((2 * H, 2 * H), lambda i: (0, 0)),
        ],
        out_specs=pl.BlockSpec((CBLK, 2 * H), lambda i: (i, 0)),
        out_shape=jax.ShapeDtypeStruct((NR, 2 * H), jnp.float32),
        compiler_params=pltpu.CompilerParams(
            fuse_transposed_lhs_in_matmul=True),
    )(embT, embT, embT, embT, eye)


def _sc_gather(idx3, table, n_ch, per_w):
    mesh = plsc.VectorSubcoreMesh(
        core_axis_name="c", subcore_axis_name="s",
        num_cores=NC, num_subcores=NS)

    @functools.partial(
        pl.kernel,
        out_type=jax.ShapeDtypeStruct((NW * per_w, 128), jnp.float32),
        mesh=mesh,
        scratch_types=[
            pltpu.VMEM((n_ch, CH), jnp.int32),
            pltpu.VMEM((2, CH, 128), jnp.float32),
            pltpu.SemaphoreType.DMA,
            pltpu.SemaphoreType.DMA,
        ],
    )
    def body(idx_hbm, table_hbm, out_hbm, idx_v, rows_v, sem_g, sem_w):
        wid = lax.axis_index("s") * NC + lax.axis_index("c")
        base = wid * per_w
        pltpu.sync_copy(idx_hbm.at[wid], idx_v)
        writes = [None, None]
        for j in range(n_ch):
            s = j % 2
            if writes[s] is not None:
                writes[s].wait()
            pltpu.async_copy(
                table_hbm.at[idx_v.at[j]], rows_v.at[s], sem_g
            ).wait()
            writes[s] = pltpu.async_copy(
                rows_v.at[s], out_hbm.at[pl.ds(base + j * CH, CH)], sem_w
            )
        for w in writes:
            if w is not None:
                w.wait()

    return body(idx3, table)


def _mlp_body(ga_ref, gb_ref, fa_ref, fb_ref, wa_ref, wb_ref, b1_ref,
              w2_ref, b2_ref, o_ref):
    bb = ga_ref.shape[1]
    ge64 = lax.broadcasted_iota(jnp.int32, (bb, 128), 1) >= 64
    ones = jnp.ones((1, 128), jnp.float32)
    dn0 = (((0,), (0,)), ((), ()))

    def unpack_select(g_ref, f_ref):
        u = jax.lax.bitcast_convert_type(g_ref[0], jnp.uint32)
        lo = jax.lax.bitcast_convert_type(u << 16, jnp.float32)
        hi = jax.lax.bitcast_convert_type(u & jnp.uint32(0xFFFF0000),
                                          jnp.float32)
        f = f_ref[0]
        mp = lax.dot_general(f[0:1, :], ones, dn0,
                             preferred_element_type=jnp.float32)
        mh = lax.dot_general(f[1:2, :], ones, dn0,
                             preferred_element_type=jnp.float32)
        v = jnp.where(mp > 0.5, hi, lo)
        m = jnp.where(ge64, mh, 1.0 - mh)
        return v * m

    am = unpack_select(ga_ref, fa_ref)
    bm = unpack_select(gb_ref, fb_ref)
    was = jnp.concatenate([wa_ref[...], wa_ref[...]], axis=0)
    wbs = jnp.concatenate([wb_ref[...], wb_ref[...]], axis=0)
    h = jnp.dot(am, was, preferred_element_type=jnp.float32)
    h = h + jnp.dot(bm, wbs, preferred_element_type=jnp.float32)
    h = jnp.maximum(h + b1_ref[...], 0.0)
    o_ref[...] = (
        jnp.dot(h, w2_ref[...], preferred_element_type=jnp.float32)
        + b2_ref[...]
    )


def kernel(x, emb, W1, b1, W2, b2):
    B = x.shape[0]
    H = emb.shape[1]
    L = W2.shape[0]

    total = 2 * B
    per_w = total // NW
    n_ch = per_w // CH

    xt = x.T
    half = (xt >= NP).astype(jnp.int32)
    q = xt - OFF * half
    packhi = (q >= NR).astype(jnp.int32)
    idx3 = (q - P2 * packhi).reshape(NW, n_ch, CH)
    flags = jnp.stack(
        [packhi.astype(jnp.float32), half.astype(jnp.float32)], axis=1
    )

    embT = emb.T
    Q = _repack(embT)
    g = _sc_gather(idx3, Q, n_ch, per_w)
    g3 = g.reshape(2, B, 2 * H)

    Wa = W1[:, :H].T
    Wb = W1[:, H:].T
    W2T = W2.T

    BB = 4096
    grid = (B // BB,)
    out = pl.pallas_call(
        _mlp_body,
        grid=grid,
        in_specs=[
            pl.BlockSpec((1, BB, 2 * H), lambda i: (0, i, 0)),
            pl.BlockSpec((1, BB, 2 * H), lambda i: (1, i, 0)),
            pl.BlockSpec((1, 2, BB), lambda i: (0, 0, i)),
            pl.BlockSpec((1, 2, BB), lambda i: (1, 0, i)),
            pl.BlockSpec((H, H), lambda i: (0, 0)),
            pl.BlockSpec((H, H), lambda i: (0, 0)),
            pl.BlockSpec((1, H), lambda i: (0, 0)),
            pl.BlockSpec((H, L), lambda i: (0, 0)),
            pl.BlockSpec((1, L), lambda i: (0, 0)),
        ],
        out_specs=pl.BlockSpec((BB, L), lambda i: (i, 0)),
        out_shape=jax.ShapeDtypeStruct((B, L), jnp.float32),
    )(g3, g3, flags, flags, Wa, Wb, b1.reshape(1, H), W2T, b2.reshape(1, L))
    return out
